# baseline (speedup 1.0000x reference)
import numpy as np

import jax
import jax.numpy as jnp
from jax.experimental import pallas as pl
from jax.experimental.pallas import tpu as pltpu


def _sobel_body(x_ref, m_ref, o_ref):
    x = x_ref[...].astype(jnp.float32)
    bn, h, w = x.shape

    row = jax.lax.broadcasted_iota(jnp.int32, (bn, h, w), 1)
    xu = jnp.where(row == 0, 0.0, pltpu.roll(x, 1, axis=1))
    xd = jnp.where(row == h - 1, 0.0, pltpu.roll(x, h - 1, axis=1))
    s = (xu + xd) + 2.0 * x
    d = xd - xu

    sd = jnp.concatenate(
        [s.reshape(bn * h, w), d.reshape(bn * h, w)], axis=1
    ).astype(jnp.bfloat16)
    g = jnp.dot(sd, m_ref[...], preferred_element_type=jnp.float32)
    gx = g[:, :w]
    gy = g[:, w:]

    g2 = gx * gx + gy * gy
    mag = g2 * jax.lax.rsqrt(jnp.maximum(g2, jnp.float32(1e-37)))
    o_ref[...] = mag.reshape(bn, h, w).astype(o_ref.dtype)


def _sobel_rhs(w: int) -> np.ndarray:
    a = np.eye(w, k=-1) - np.eye(w, k=1)
    b = np.eye(w, k=-1) + 2.0 * np.eye(w) + np.eye(w, k=1)
    m = np.zeros((2 * w, 2 * w), dtype=np.float32)
    m[:w, :w] = a
    m[w:, w:] = b
    return m.astype(jnp.bfloat16)


def _num_tensorcores() -> int:
    try:
        n = int(jax.devices()[0].num_cores)
        return n if n >= 1 else 1
    except Exception:
        return 1


def _sobel_call(x3, bn, ncores):
    N, H, W = x3.shape
    itemsize = jnp.dtype(x3.dtype).itemsize
    g = N // bn
    tile_bytes = bn * H * W * itemsize
    vmem_limit = int(min(56 << 20, max(8 << 20, 10 * tile_bytes)))

    rhs = jnp.asarray(_sobel_rhs(W))

    cost = pl.CostEstimate(
        flops=2 * N * H * W * 2 * W,
        transcendentals=N * H * W,
        bytes_accessed=2 * N * H * W * itemsize,
    )

    if ncores > 1 and g % ncores == 0:
        gc = g // ncores
        grid = (ncores, gc)
        in_specs = [
            pl.BlockSpec((bn, H, W), lambda c, i: (c * gc + i, 0, 0)),
            pl.BlockSpec((2 * W, 2 * W), lambda c, i: (0, 0)),
        ]
        out_specs = pl.BlockSpec((bn, H, W), lambda c, i: (c * gc + i, 0, 0))
        semantics = ("core_parallel", "arbitrary")
    else:
        grid = (g,)
        in_specs = [
            pl.BlockSpec((bn, H, W), lambda i: (i, 0, 0)),
            pl.BlockSpec((2 * W, 2 * W), lambda i: (0, 0)),
        ]
        out_specs = pl.BlockSpec((bn, H, W), lambda i: (i, 0, 0))
        semantics = ("arbitrary",)

    return pl.pallas_call(
        _sobel_body,
        out_shape=jax.ShapeDtypeStruct((N, H, W), x3.dtype),
        grid=grid,
        in_specs=in_specs,
        out_specs=out_specs,
        compiler_params=pltpu.CompilerParams(
            dimension_semantics=semantics,
            vmem_limit_bytes=vmem_limit,
        ),
        cost_estimate=cost,
    )(x3, rhs)


def kernel(x):
    N, C, H, W = x.shape
    assert C == 1

    x3 = x.reshape(N, H, W)
    ncores = _num_tensorcores()

    itemsize = jnp.dtype(x.dtype).itemsize
    per_image = H * W * itemsize
    target = 4 << 20
    bn = 1
    for cand in (32, 64, 16, 8, 4, 2, 1):
        steps = N // cand
        if N % cand == 0 and cand * per_image <= target and steps % max(ncores, 1) == 0:
            bn = cand
            break

    out3 = _sobel_call(x3, bn, ncores)
    return out3.reshape(N, 1, H, W)


# speedup vs baseline: 1.2072x; 1.2072x over previous
import numpy as np

import jax
import jax.numpy as jnp
from jax.experimental import pallas as pl
from jax.experimental.pallas import tpu as pltpu


def _sobel_body(x_ref, m_ref, o_ref):
    x = x_ref[...].astype(jnp.float32)
    bn, h, w = x.shape

    row = jax.lax.broadcasted_iota(jnp.int32, (bn, h, w), 1)
    xu = jnp.where(row == 0, 0.0, pltpu.roll(x, 1, axis=1))
    xd = jnp.where(row == h - 1, 0.0, pltpu.roll(x, h - 1, axis=1))
    s = (xu + xd) + 2.0 * x
    d = xd - xu

    sd = jnp.concatenate(
        [s.reshape(bn * h, w), d.reshape(bn * h, w)], axis=1
    ).astype(jnp.bfloat16)
    g = jnp.dot(sd, m_ref[...], preferred_element_type=jnp.float32)
    gx = g[:, :w]
    gy = g[:, w:]

    g2 = gx * gx + gy * gy
    mag = g2 * jax.lax.rsqrt(jnp.maximum(g2, jnp.float32(1e-37)))
    o_ref[...] = mag.reshape(bn, h, w).astype(o_ref.dtype)


def _sobel_rhs(w: int) -> np.ndarray:
    a = np.eye(w, k=-1) - np.eye(w, k=1)
    b = np.eye(w, k=-1) + 2.0 * np.eye(w) + np.eye(w, k=1)
    m = np.zeros((2 * w, 2 * w), dtype=np.float32)
    m[:w, :w] = a
    m[w:, w:] = b
    return m.astype(jnp.bfloat16)


def _num_tensorcores() -> int:
    try:
        n = int(jax.devices()[0].num_cores)
        return n if n >= 1 else 1
    except Exception:
        return 1


def _sobel_call(x3, bn, ncores):
    N, H, W = x3.shape
    itemsize = jnp.dtype(x3.dtype).itemsize
    g = N // bn
    tile_bytes = bn * H * W * itemsize
    vmem_limit = int(min(56 << 20, max(8 << 20, 10 * tile_bytes)))

    rhs = jnp.asarray(_sobel_rhs(W))

    cost = pl.CostEstimate(
        flops=2 * N * H * W * 2 * W,
        transcendentals=N * H * W,
        bytes_accessed=2 * N * H * W * itemsize,
    )

    if ncores > 1 and g % ncores == 0:
        gc = g // ncores
        grid = (ncores, gc)
        in_specs = [
            pl.BlockSpec((bn, H, W), lambda c, i: (c * gc + i, 0, 0)),
            pl.BlockSpec((2 * W, 2 * W), lambda c, i: (0, 0)),
        ]
        out_specs = pl.BlockSpec((bn, H, W), lambda c, i: (c * gc + i, 0, 0))
        semantics = ("core_parallel", "arbitrary")
    else:
        grid = (g,)
        in_specs = [
            pl.BlockSpec((bn, H, W), lambda i: (i, 0, 0)),
            pl.BlockSpec((2 * W, 2 * W), lambda i: (0, 0)),
        ]
        out_specs = pl.BlockSpec((bn, H, W), lambda i: (i, 0, 0))
        semantics = ("arbitrary",)

    return pl.pallas_call(
        _sobel_body,
        out_shape=jax.ShapeDtypeStruct((N, H, W), x3.dtype),
        grid=grid,
        in_specs=in_specs,
        out_specs=out_specs,
        compiler_params=pltpu.CompilerParams(
            dimension_semantics=semantics,
            vmem_limit_bytes=vmem_limit,
        ),
        cost_estimate=cost,
    )(x3, rhs)


def kernel(x):
    N, C, H, W = x.shape
    assert C == 1

    x3 = x.reshape(N, H, W)
    ncores = _num_tensorcores()

    itemsize = jnp.dtype(x.dtype).itemsize
    per_image = H * W * itemsize
    target = 8 << 20
    bn = 1
    for cand in (128, 64, 32, 16, 8, 4, 2, 1):
        steps = N // cand
        if N % cand == 0 and cand * per_image <= target and steps % max(ncores, 1) == 0:
            bn = cand
            break

    out3 = _sobel_call(x3, bn, ncores)
    return out3.reshape(N, 1, H, W)


# speedup vs baseline: 1.2111x; 1.0032x over previous
import numpy as np

import jax
import jax.numpy as jnp
from jax.experimental import pallas as pl
from jax.experimental.pallas import tpu as pltpu


def _sobel_body(x_ref, m_ref, o_ref):
    x = x_ref[...].astype(jnp.float32)
    bn, h, w = x.shape

    row = jax.lax.broadcasted_iota(jnp.int32, (bn, h, w), 1)
    xu = jnp.where(row == 0, 0.0, pltpu.roll(x, 1, axis=1))
    xd = jnp.where(row == h - 1, 0.0, pltpu.roll(x, h - 1, axis=1))
    s = (xu + xd) + 2.0 * x
    d = xd - xu

    sd = jnp.concatenate(
        [s.reshape(bn * h, w), d.reshape(bn * h, w)], axis=1
    ).astype(jnp.bfloat16)
    g = jnp.dot(sd, m_ref[...], preferred_element_type=jnp.float32)
    gx = g[:, :w]
    gy = g[:, w:]

    g2 = gx * gx + gy * gy
    mag = g2 * jax.lax.rsqrt(jnp.maximum(g2, jnp.float32(1e-37)))
    o_ref[...] = mag.reshape(bn, h, w).astype(o_ref.dtype)


def _sobel_rhs(w: int) -> np.ndarray:
    a = np.eye(w, k=-1) - np.eye(w, k=1)
    b = np.eye(w, k=-1) + 2.0 * np.eye(w) + np.eye(w, k=1)
    m = np.zeros((2 * w, 2 * w), dtype=np.float32)
    m[:w, :w] = a
    m[w:, w:] = b
    return m.astype(jnp.bfloat16)


def _sobel_call(x3, bn):
    N, H, W = x3.shape
    itemsize = jnp.dtype(x3.dtype).itemsize
    g = N // bn
    tile_bytes = bn * H * W * itemsize
    vmem_limit = int(min(56 << 20, max(8 << 20, 10 * tile_bytes)))

    rhs = jnp.asarray(_sobel_rhs(W))

    cost = pl.CostEstimate(
        flops=2 * N * H * W * 2 * W,
        transcendentals=N * H * W,
        bytes_accessed=2 * N * H * W * itemsize,
    )

    return pl.pallas_call(
        _sobel_body,
        out_shape=jax.ShapeDtypeStruct((N, H, W), x3.dtype),
        grid=(g,),
        in_specs=[
            pl.BlockSpec((bn, H, W), lambda i: (i, 0, 0)),
            pl.BlockSpec((2 * W, 2 * W), lambda i: (0, 0)),
        ],
        out_specs=pl.BlockSpec((bn, H, W), lambda i: (i, 0, 0)),
        compiler_params=pltpu.CompilerParams(
            dimension_semantics=("arbitrary",),
            vmem_limit_bytes=vmem_limit,
        ),
        cost_estimate=cost,
    )(x3, rhs)


def kernel(x):
    N, C, H, W = x.shape
    assert C == 1

    x3 = x.reshape(N, H, W)

    itemsize = jnp.dtype(x.dtype).itemsize
    per_image = H * W * itemsize
    target = 8 << 20
    bn = 1
    for cand in (128, 64, 32, 16, 8, 4, 2, 1):
        if N % cand == 0 and cand * per_image <= target:
            bn = cand
            break

    out3 = _sobel_call(x3, bn)
    return out3.reshape(N, 1, H, W)
